# baseline (speedup 1.0000x reference)
import functools
import math

import jax
import jax.numpy as jnp
from jax.experimental import pallas as pl
from jax.experimental.pallas import tpu as pltpu

try:
    _CONST = pl.Buffered(1)
except Exception:
    _CONST = None


def _layer_norm(z, g, b, inv_d, eps):
    mean = jnp.sum(z, axis=-1, keepdims=True) * inv_d
    var = jnp.sum(z * z, axis=-1, keepdims=True) * inv_d - mean * mean
    rstd = jax.lax.rsqrt(jnp.maximum(var, 0.0) + eps)
    return (z - mean) * rstd * g + b


def _mha_heads(q, k, v, bias, num_heads, scale):
    d = q.shape[-1]
    dk = d // num_heads
    outs = []
    for h in range(num_heads):
        sl = slice(h * dk, (h + 1) * dk)
        s = jax.lax.dot_general(q[:, sl], k[:, sl], (((1,), (1,)), ((), ())),
                                preferred_element_type=jnp.float32)
        s = s * scale
        if bias is not None:
            s = s + bias
        m = jnp.max(s, axis=-1, keepdims=True)
        p = jnp.exp(s - m)
        p = p / jnp.sum(p, axis=-1, keepdims=True)
        outs.append(jnp.dot(p.astype(jnp.bfloat16), v[:, sl],
                            preferred_element_type=jnp.float32))
    return jnp.concatenate(outs, axis=-1)


def _decoder_kernel(x_ref, mem_ref, mask_ref,
                    wqkv1_ref, bqkv1_ref, wo1_ref, bo1_ref,
                    wq2_ref, bq2_ref, wkv2_ref, bkv2_ref, wo2_ref, bo2_ref,
                    ln1g_ref, ln1b_ref, ln2g_ref, ln2b_ref, ln3g_ref, ln3b_ref,
                    w1_ref, b1_ref, w2_ref, b2_ref,
                    o_ref, *, num_heads, scale, inv_d, eps):
    bf16 = jnp.bfloat16
    x = x_ref[0]
    d = x.shape[-1]

    qkv = jnp.dot(x.astype(bf16), wqkv1_ref[...],
                  preferred_element_type=jnp.float32) + bqkv1_ref[...]
    qkv = qkv.astype(bf16)
    bias = jnp.where(mask_ref[0] == 0, -1e9, 0.0).astype(jnp.float32)
    attn = _mha_heads(qkv[:, :d], qkv[:, d:2 * d], qkv[:, 2 * d:],
                      bias, num_heads, scale)
    y = jnp.dot(attn.astype(bf16), wo1_ref[...],
                preferred_element_type=jnp.float32) + bo1_ref[...]
    x2 = _layer_norm(x + y, ln1g_ref[...], ln1b_ref[...], inv_d, eps)

    q2 = jnp.dot(x2.astype(bf16), wq2_ref[...],
                 preferred_element_type=jnp.float32) + bq2_ref[...]
    kv2 = jnp.dot(mem_ref[0].astype(bf16), wkv2_ref[...],
                  preferred_element_type=jnp.float32) + bkv2_ref[...]
    q2 = q2.astype(bf16)
    kv2 = kv2.astype(bf16)
    attn2 = _mha_heads(q2, kv2[:, :d], kv2[:, d:], None, num_heads, scale)
    x3 = jnp.dot(attn2.astype(bf16), wo2_ref[...],
                 preferred_element_type=jnp.float32) + bo2_ref[...]
    z = _layer_norm(x3 + x3, ln2g_ref[...], ln2b_ref[...], inv_d, eps)

    h = jnp.maximum(jnp.dot(z.astype(bf16), w1_ref[...],
                            preferred_element_type=jnp.float32) + b1_ref[...],
                    0.0)
    yf = jnp.dot(h.astype(bf16), w2_ref[...],
                 preferred_element_type=jnp.float32) + b2_ref[...]
    o_ref[0] = _layer_norm(z + yf, ln3g_ref[...], ln3b_ref[...],
                           inv_d, eps).astype(o_ref.dtype)


def kernel(x, kv_memory, tgt_mask,
           attn1_wq, attn1_bq, attn1_wk, attn1_bk,
           attn1_wv, attn1_bv, attn1_wo, attn1_bo,
           attn2_wq, attn2_bq, attn2_wk, attn2_bk,
           attn2_wv, attn2_bv, attn2_wo, attn2_bo,
           ln1_g, ln1_b, ln2_g, ln2_b, ln3_g, ln3_b,
           w1, b1, w2, b2):
    B, S, d_model = x.shape
    num_heads = 4
    d_k = d_model // num_heads
    f = w1.shape[1]
    dt = x.dtype

    bf16 = jnp.bfloat16
    wqkv1 = jnp.concatenate([attn1_wq, attn1_wk, attn1_wv], axis=1).astype(bf16)
    bqkv1 = jnp.concatenate([attn1_bq, attn1_bk, attn1_bv]).reshape(1, 3 * d_model)
    wkv2 = jnp.concatenate([attn2_wk, attn2_wv], axis=1).astype(bf16)
    bkv2 = jnp.concatenate([attn2_bk, attn2_bv]).reshape(1, 2 * d_model)

    def row(v):
        return v.reshape(1, -1)

    const2d = lambda r, c: pl.BlockSpec((r, c), lambda b: (0, 0),
                                        pipeline_mode=_CONST)

    kern = functools.partial(_decoder_kernel, num_heads=num_heads,
                             scale=1.0 / math.sqrt(d_k), inv_d=1.0 / d_model,
                             eps=1e-5)

    flops = (2 * B * S * d_model * (3 * d_model)
             + 2 * B * S * S * d_k * num_heads * 2
             + 2 * B * S * d_model * d_model
             + 2 * B * S * d_model * (3 * d_model)
             + 2 * B * S * S * d_k * num_heads * 2
             + 2 * B * S * d_model * d_model
             + 4 * B * S * d_model * f)
    nbytes = jnp.dtype(dt).itemsize
    weight_bytes = (d_model * (8 * d_model + 2 * f)) * nbytes
    cost = pl.CostEstimate(
        flops=flops, transcendentals=2 * B * S * S * num_heads,
        bytes_accessed=(3 * B * S * d_model + B * S * S) * nbytes + weight_bytes)

    out = pl.pallas_call(
        kern,
        out_shape=jax.ShapeDtypeStruct((B, S, d_model), dt),
        grid_spec=pltpu.PrefetchScalarGridSpec(
            num_scalar_prefetch=0,
            grid=(B,),
            in_specs=[
                pl.BlockSpec((1, S, d_model), lambda b: (b, 0, 0)),
                pl.BlockSpec((1, S, d_model), lambda b: (b, 0, 0)),
                pl.BlockSpec((1, S, S), lambda b: (b, 0, 0)),
                const2d(d_model, 3 * d_model), const2d(1, 3 * d_model),
                const2d(d_model, d_model), const2d(1, d_model),
                const2d(d_model, d_model), const2d(1, d_model),
                const2d(d_model, 2 * d_model), const2d(1, 2 * d_model),
                const2d(d_model, d_model), const2d(1, d_model),
                const2d(1, d_model), const2d(1, d_model),
                const2d(1, d_model), const2d(1, d_model),
                const2d(1, d_model), const2d(1, d_model),
                const2d(d_model, f), const2d(1, f),
                const2d(f, d_model), const2d(1, d_model),
            ],
            out_specs=pl.BlockSpec((1, S, d_model), lambda b: (b, 0, 0)),
        ),
        compiler_params=pltpu.CompilerParams(
            dimension_semantics=("parallel",)),
        cost_estimate=cost,
    )(x, kv_memory, tgt_mask,
      wqkv1, bqkv1, attn1_wo.astype(bf16), row(attn1_bo),
      attn2_wq.astype(bf16), row(attn2_bq), wkv2, bkv2,
      attn2_wo.astype(bf16), row(attn2_bo),
      row(ln1_g), row(ln1_b), row(ln2_g), row(ln2_b), row(ln3_g), row(ln3_b),
      w1.astype(bf16), row(b1), w2.astype(bf16), row(b2))
    return out


# speedup vs baseline: 1.1352x; 1.1352x over previous
import functools
import math

import jax
import jax.numpy as jnp
from jax.experimental import pallas as pl
from jax.experimental.pallas import tpu as pltpu

try:
    _CONST = pl.Buffered(1)
except Exception:
    _CONST = None


def _layer_norm(z, g, b, inv_d, eps):
    mean = jnp.sum(z, axis=-1, keepdims=True) * inv_d
    var = jnp.sum(z * z, axis=-1, keepdims=True) * inv_d - mean * mean
    rstd = jax.lax.rsqrt(jnp.maximum(var, 0.0) + eps)
    return (z - mean) * rstd * g + b


def _mha_heads(q, k, v, bias, num_heads, scale):
    d = q.shape[-1]
    dk = d // num_heads
    outs = []
    for h in range(num_heads):
        sl = slice(h * dk, (h + 1) * dk)
        s = jax.lax.dot_general(q[:, sl], k[:, sl], (((1,), (1,)), ((), ())),
                                preferred_element_type=jnp.float32)
        s = s * scale
        if bias is not None:
            s = s + bias
        m = jnp.max(s, axis=-1, keepdims=True)
        p = jnp.exp(s - m)
        r = 1.0 / jnp.sum(p, axis=-1, keepdims=True)
        outs.append(jnp.dot(p, v[:, sl],
                            preferred_element_type=jnp.float32) * r)
    return jnp.concatenate(outs, axis=-1)


def _decoder_kernel(x_ref, mem_ref, mask_ref,
                    wqkv1_ref, bqkv1_ref, wo1_ref, bo1_ref,
                    wq2_ref, bq2_ref, wkv2_ref, bkv2_ref, wo2_ref, bo2_ref,
                    ln1g_ref, ln1b_ref, ln2g_ref, ln2b_ref, ln3g_ref, ln3b_ref,
                    w1_ref, b1_ref, w2_ref, b2_ref,
                    o_ref, *, num_heads, scale, inv_d, eps):
    x = x_ref[0]
    d = x.shape[-1]

    qkv = jnp.dot(x, wqkv1_ref[...],
                  preferred_element_type=jnp.float32) + bqkv1_ref[...]
    bias = jnp.where(mask_ref[0] == 0, -1e9, 0.0).astype(jnp.float32)
    attn = _mha_heads(qkv[:, :d], qkv[:, d:2 * d], qkv[:, 2 * d:],
                      bias, num_heads, scale)
    y = jnp.dot(attn, wo1_ref[...],
                preferred_element_type=jnp.float32) + bo1_ref[...]
    x2 = _layer_norm(x + y, ln1g_ref[...], ln1b_ref[...], inv_d, eps)

    q2 = jnp.dot(x2, wq2_ref[...],
                 preferred_element_type=jnp.float32) + bq2_ref[...]
    kv2 = jnp.dot(mem_ref[0], wkv2_ref[...],
                  preferred_element_type=jnp.float32) + bkv2_ref[...]
    attn2 = _mha_heads(q2, kv2[:, :d], kv2[:, d:], None, num_heads, scale)
    x3 = jnp.dot(attn2, wo2_ref[...],
                 preferred_element_type=jnp.float32) + bo2_ref[...]
    z = _layer_norm(x3 + x3, ln2g_ref[...], ln2b_ref[...], inv_d, eps)

    h = jnp.maximum(jnp.dot(z, w1_ref[...],
                            preferred_element_type=jnp.float32) + b1_ref[...],
                    0.0)
    yf = jnp.dot(h, w2_ref[...],
                 preferred_element_type=jnp.float32) + b2_ref[...]
    o_ref[0] = _layer_norm(z + yf, ln3g_ref[...], ln3b_ref[...],
                           inv_d, eps).astype(o_ref.dtype)


def kernel(x, kv_memory, tgt_mask,
           attn1_wq, attn1_bq, attn1_wk, attn1_bk,
           attn1_wv, attn1_bv, attn1_wo, attn1_bo,
           attn2_wq, attn2_bq, attn2_wk, attn2_bk,
           attn2_wv, attn2_bv, attn2_wo, attn2_bo,
           ln1_g, ln1_b, ln2_g, ln2_b, ln3_g, ln3_b,
           w1, b1, w2, b2):
    B, S, d_model = x.shape
    num_heads = 4
    d_k = d_model // num_heads
    f = w1.shape[1]
    dt = x.dtype

    wqkv1 = jnp.concatenate([attn1_wq, attn1_wk, attn1_wv], axis=1)
    bqkv1 = jnp.concatenate([attn1_bq, attn1_bk, attn1_bv]).reshape(1, 3 * d_model)
    wkv2 = jnp.concatenate([attn2_wk, attn2_wv], axis=1)
    bkv2 = jnp.concatenate([attn2_bk, attn2_bv]).reshape(1, 2 * d_model)

    def row(v):
        return v.reshape(1, -1)

    const2d = lambda r, c: pl.BlockSpec((r, c), lambda b: (0, 0),
                                        pipeline_mode=_CONST)

    kern = functools.partial(_decoder_kernel, num_heads=num_heads,
                             scale=1.0 / math.sqrt(d_k), inv_d=1.0 / d_model,
                             eps=1e-5)

    flops = (2 * B * S * d_model * (3 * d_model)
             + 2 * B * S * S * d_k * num_heads * 2
             + 2 * B * S * d_model * d_model
             + 2 * B * S * d_model * (3 * d_model)
             + 2 * B * S * S * d_k * num_heads * 2
             + 2 * B * S * d_model * d_model
             + 4 * B * S * d_model * f)
    nbytes = jnp.dtype(dt).itemsize
    weight_bytes = (d_model * (8 * d_model + 2 * f)) * nbytes
    cost = pl.CostEstimate(
        flops=flops, transcendentals=2 * B * S * S * num_heads,
        bytes_accessed=(3 * B * S * d_model + B * S * S) * nbytes + weight_bytes)

    out = pl.pallas_call(
        kern,
        out_shape=jax.ShapeDtypeStruct((B, S, d_model), dt),
        grid_spec=pltpu.PrefetchScalarGridSpec(
            num_scalar_prefetch=0,
            grid=(B,),
            in_specs=[
                pl.BlockSpec((1, S, d_model), lambda b: (b, 0, 0)),
                pl.BlockSpec((1, S, d_model), lambda b: (b, 0, 0)),
                pl.BlockSpec((1, S, S), lambda b: (b, 0, 0)),
                const2d(d_model, 3 * d_model), const2d(1, 3 * d_model),
                const2d(d_model, d_model), const2d(1, d_model),
                const2d(d_model, d_model), const2d(1, d_model),
                const2d(d_model, 2 * d_model), const2d(1, 2 * d_model),
                const2d(d_model, d_model), const2d(1, d_model),
                const2d(1, d_model), const2d(1, d_model),
                const2d(1, d_model), const2d(1, d_model),
                const2d(1, d_model), const2d(1, d_model),
                const2d(d_model, f), const2d(1, f),
                const2d(f, d_model), const2d(1, d_model),
            ],
            out_specs=pl.BlockSpec((1, S, d_model), lambda b: (b, 0, 0)),
        ),
        compiler_params=pltpu.CompilerParams(
            dimension_semantics=("parallel",)),
        cost_estimate=cost,
    )(x, kv_memory, tgt_mask,
      wqkv1, bqkv1, attn1_wo, row(attn1_bo),
      attn2_wq, row(attn2_bq), wkv2, bkv2, attn2_wo, row(attn2_bo),
      row(ln1_g), row(ln1_b), row(ln2_g), row(ln2_b), row(ln3_g), row(ln3_b),
      w1, row(b1), w2, row(b2))
    return out


# speedup vs baseline: 1.1697x; 1.0304x over previous
import functools
import math

import jax
import jax.numpy as jnp
from jax.experimental import pallas as pl
from jax.experimental.pallas import tpu as pltpu

try:
    _CONST = pl.Buffered(1)
except Exception:
    _CONST = None


def _layer_norm(z, g, b, inv_d, eps):
    mean = jnp.sum(z, axis=-1, keepdims=True) * inv_d
    var = jnp.sum(z * z, axis=-1, keepdims=True) * inv_d - mean * mean
    rstd = jax.lax.rsqrt(jnp.maximum(var, 0.0) + eps)
    return (z - mean) * rstd * g + b


def _mha_heads(q, k, v, bias, num_heads):
    d = q.shape[-1]
    dk = d // num_heads
    outs = []
    for h in range(num_heads):
        sl = slice(h * dk, (h + 1) * dk)
        s = jax.lax.dot_general(q[:, sl], k[:, sl], (((1,), (1,)), ((), ())),
                                preferred_element_type=jnp.float32)
        if bias is not None:
            s = s + bias
        m = jnp.max(s, axis=-1, keepdims=True)
        p = jnp.exp(s - m)
        r = 1.0 / jnp.sum(p, axis=-1, keepdims=True)
        outs.append(jnp.dot(p, v[:, sl],
                            preferred_element_type=jnp.float32) * r)
    return jnp.concatenate(outs, axis=-1)


def _decoder_kernel(x_ref, mem_ref, mask_ref,
                    wqkv1_ref, bqkv1_ref, wo1_ref, bo1_ref,
                    wq2_ref, bq2_ref, wkv2_ref, bkv2_ref, wo2_ref, bo2_ref,
                    ln1g_ref, ln1b_ref, ln2g_ref, ln2b_ref, ln3g_ref, ln3b_ref,
                    w1_ref, b1_ref, w2_ref, b2_ref,
                    o_ref, *, num_heads, bpg, inv_d, eps):
    nb, S, d = x_ref.shape
    x = x_ref[...].reshape(nb * S, d)

    qkv = jnp.dot(x, wqkv1_ref[...],
                  preferred_element_type=jnp.float32) + bqkv1_ref[...]
    attns = []
    for j in range(bpg):
        rows = slice(j * S, (j + 1) * S)
        bias = jnp.where(mask_ref[j] == 0, -1e9, 0.0).astype(jnp.float32)
        attns.append(_mha_heads(qkv[rows, :d], qkv[rows, d:2 * d],
                                qkv[rows, 2 * d:], bias, num_heads))
    attn = jnp.concatenate(attns, axis=0)
    y = jnp.dot(attn, wo1_ref[...],
                preferred_element_type=jnp.float32) + bo1_ref[...]
    x2 = _layer_norm(x + y, ln1g_ref[...], ln1b_ref[...], inv_d, eps)

    q2 = jnp.dot(x2, wq2_ref[...],
                 preferred_element_type=jnp.float32) + bq2_ref[...]
    kv2 = jnp.dot(mem_ref[...].reshape(nb * S, d), wkv2_ref[...],
                  preferred_element_type=jnp.float32) + bkv2_ref[...]
    attns = []
    for j in range(bpg):
        rows = slice(j * S, (j + 1) * S)
        attns.append(_mha_heads(q2[rows], kv2[rows, :d], kv2[rows, d:],
                                None, num_heads))
    attn2 = jnp.concatenate(attns, axis=0)
    x3 = jnp.dot(attn2, wo2_ref[...],
                 preferred_element_type=jnp.float32) + bo2_ref[...]
    z = _layer_norm(x3 + x3, ln2g_ref[...], ln2b_ref[...], inv_d, eps)

    h = jnp.maximum(jnp.dot(z, w1_ref[...],
                            preferred_element_type=jnp.float32) + b1_ref[...],
                    0.0)
    yf = jnp.dot(h, w2_ref[...],
                 preferred_element_type=jnp.float32) + b2_ref[...]
    o_ref[...] = _layer_norm(z + yf, ln3g_ref[...], ln3b_ref[...],
                             inv_d, eps).astype(o_ref.dtype).reshape(nb, S, d)


def kernel(x, kv_memory, tgt_mask,
           attn1_wq, attn1_bq, attn1_wk, attn1_bk,
           attn1_wv, attn1_bv, attn1_wo, attn1_bo,
           attn2_wq, attn2_bq, attn2_wk, attn2_bk,
           attn2_wv, attn2_bv, attn2_wo, attn2_bo,
           ln1_g, ln1_b, ln2_g, ln2_b, ln3_g, ln3_b,
           w1, b1, w2, b2):
    B, S, d_model = x.shape
    num_heads = 4
    d_k = d_model // num_heads
    f = w1.shape[1]
    dt = x.dtype
    bpg = 2 if B % 2 == 0 else 1
    scale = 1.0 / math.sqrt(d_k)

    wqkv1 = jnp.concatenate([attn1_wq * scale, attn1_wk, attn1_wv], axis=1)
    bqkv1 = jnp.concatenate([attn1_bq * scale, attn1_bk,
                             attn1_bv]).reshape(1, 3 * d_model)
    wkv2 = jnp.concatenate([attn2_wk, attn2_wv], axis=1)
    bkv2 = jnp.concatenate([attn2_bk, attn2_bv]).reshape(1, 2 * d_model)
    wq2s = attn2_wq * scale
    bq2s = (attn2_bq * scale).reshape(1, d_model)

    def row(v):
        return v.reshape(1, -1)

    const2d = lambda r, c: pl.BlockSpec((r, c), lambda i: (0, 0),
                                        pipeline_mode=_CONST)

    kern = functools.partial(_decoder_kernel, num_heads=num_heads, bpg=bpg,
                             inv_d=1.0 / d_model, eps=1e-5)

    flops = (2 * B * S * d_model * (3 * d_model)
             + 2 * B * S * S * d_k * num_heads * 2
             + 2 * B * S * d_model * d_model
             + 2 * B * S * d_model * (3 * d_model)
             + 2 * B * S * S * d_k * num_heads * 2
             + 2 * B * S * d_model * d_model
             + 4 * B * S * d_model * f)
    nbytes = jnp.dtype(dt).itemsize
    weight_bytes = (d_model * (8 * d_model + 2 * f)) * nbytes
    cost = pl.CostEstimate(
        flops=flops, transcendentals=2 * B * S * S * num_heads,
        bytes_accessed=(3 * B * S * d_model + B * S * S) * nbytes + weight_bytes)

    out = pl.pallas_call(
        kern,
        out_shape=jax.ShapeDtypeStruct((B, S, d_model), dt),
        grid_spec=pltpu.PrefetchScalarGridSpec(
            num_scalar_prefetch=0,
            grid=(B // bpg,),
            in_specs=[
                pl.BlockSpec((bpg, S, d_model), lambda i: (i, 0, 0)),
                pl.BlockSpec((bpg, S, d_model), lambda i: (i, 0, 0)),
                pl.BlockSpec((bpg, S, S), lambda i: (i, 0, 0)),
                const2d(d_model, 3 * d_model), const2d(1, 3 * d_model),
                const2d(d_model, d_model), const2d(1, d_model),
                const2d(d_model, d_model), const2d(1, d_model),
                const2d(d_model, 2 * d_model), const2d(1, 2 * d_model),
                const2d(d_model, d_model), const2d(1, d_model),
                const2d(1, d_model), const2d(1, d_model),
                const2d(1, d_model), const2d(1, d_model),
                const2d(1, d_model), const2d(1, d_model),
                const2d(d_model, f), const2d(1, f),
                const2d(f, d_model), const2d(1, d_model),
            ],
            out_specs=pl.BlockSpec((bpg, S, d_model), lambda i: (i, 0, 0)),
        ),
        compiler_params=pltpu.CompilerParams(
            dimension_semantics=("parallel",)),
        cost_estimate=cost,
    )(x, kv_memory, tgt_mask,
      wqkv1, bqkv1, attn1_wo, row(attn1_bo),
      wq2s, bq2s, wkv2, bkv2, attn2_wo, row(attn2_bo),
      row(ln1_g), row(ln1_b), row(ln2_g), row(ln2_b), row(ln3_g), row(ln3_b),
      w1, row(b1), w2, row(b2))
    return out
